# Initial kernel scaffold; baseline (speedup 1.0000x reference)
import jax
import jax.numpy as jnp
from jax.experimental import pallas as pl


def kernel(pos, edge_index, batch, t, z, t_W1, t_b1, t_W2, t_b2, c_W, c_b, We1, be1, We2, be2, Wx1, bx1, Wx2, bx2, Wh1, bh1, Wh2, bh2, Wout, bout):
    raise NotImplementedError("write your pallas kernel here")


# speedup vs baseline: 1.9127x; 1.9127x over previous
import functools

import jax
import jax.numpy as jnp
from jax import lax
from jax.experimental import pallas as pl
from jax.experimental.pallas import tpu as pltpu
from jax.experimental.pallas import tpu_sc as plsc

V = 10000
E = 320000
B = 8
HD = 128
LD = 64
TD = 16
NL = 4

VP = 10240
EP = 327680
DW = 144
DX = 16
NW = 32
EPW = EP // NW
CH = 128
NCH = EPW // CH
RPT = VP // 16

EB = 2048
VB = 1024

F32 = jnp.float32


def _silu(x):
    return x * jax.nn.sigmoid(x)


_SC_MESH = plsc.VectorSubcoreMesh(
    core_axis_name="c", subcore_axis_name="s", num_cores=2, num_subcores=16)


def _sc_gather_body(gd, gs, dstp, srcp, out, didx, sidx, buf_a, buf_b, sem1, sem2):
    c = lax.axis_index("c")
    s = lax.axis_index("s")
    wid = s * 2 + c
    base_w = wid * EPW

    def chunk(i, carry):
        base = base_w + i * CH
        pltpu.sync_copy(dstp.at[pl.ds(base, CH)], didx)
        pltpu.sync_copy(srcp.at[pl.ds(base, CH)], sidx)
        cp1 = pltpu.async_copy(gd.at[didx], buf_a, sem1)
        cp2 = pltpu.async_copy(gs.at[sidx], buf_b, sem2)
        cp1.wait()
        cp2.wait()

        def addrow(r, cc):
            for j in range(DW // 16):
                sl = pl.ds(j * 16, 16)
                buf_a[r, sl] = buf_a[r, sl] + buf_b[r, sl]
            return cc

        lax.fori_loop(0, CH, addrow, 0)
        pltpu.sync_copy(buf_a, out.at[pl.ds(base, CH)])
        return carry

    lax.fori_loop(0, NCH, chunk, 0)


_SC_PARAMS = pltpu.CompilerParams(use_tc_tiling_on_sc=False)

_sc_gather = pl.kernel(
    _sc_gather_body,
    out_type=jax.ShapeDtypeStruct((EP, DW), F32),
    mesh=_SC_MESH,
    compiler_params=_SC_PARAMS,
    scratch_types=[
        pltpu.VMEM((CH,), jnp.int32),
        pltpu.VMEM((CH,), jnp.int32),
        pltpu.VMEM((CH, DW), F32),
        pltpu.VMEM((CH, DW), F32),
        pltpu.SemaphoreType.DMA,
        pltpu.SemaphoreType.DMA,
    ],
)


def _sc_scatter_body(dw, w_hbm, dstp, out, idxb, buf, zbuf, acc):
    c = lax.axis_index("c")
    s = lax.axis_index("s")
    wid = s * 2 + c
    base_w = wid * EPW

    def zrow(r, cc):
        for j in range(dw // 16):
            zbuf[r, pl.ds(j * 16, 16)] = jnp.zeros((16,), F32)
        return cc

    lax.fori_loop(0, CH, zrow, 0)
    for k in range(RPT // CH):
        pltpu.sync_copy(zbuf, acc.at[pl.ds(s * RPT + k * CH, CH)])
    plsc.subcore_barrier()

    def chunk(i, carry):
        base = base_w + i * CH
        pltpu.sync_copy(dstp.at[pl.ds(base, CH)], idxb)
        pltpu.sync_copy(w_hbm.at[pl.ds(base, CH)], buf)
        pltpu.sync_copy(buf, acc.at[idxb], add=True)
        return carry

    lax.fori_loop(0, NCH, chunk, 0)
    plsc.subcore_barrier()
    for k in range(RPT // CH):
        r0 = s * RPT + k * CH
        pltpu.sync_copy(acc.at[pl.ds(r0, CH)], buf)
        pltpu.sync_copy(buf, out.at[c, pl.ds(r0, CH)])


def _make_scatter(dw):
    return pl.kernel(
        functools.partial(_sc_scatter_body, dw),
        out_type=jax.ShapeDtypeStruct((2, VP, dw), F32),
        mesh=_SC_MESH,
        compiler_params=_SC_PARAMS,
        scratch_types=[
            pltpu.VMEM((CH,), jnp.int32),
            pltpu.VMEM((CH, dw), F32),
            pltpu.VMEM((CH, dw), F32),
            pltpu.VMEM_SHARED((VP, dw), F32),
        ],
    )


_sc_scatter_full = _make_scatter(DW)
_sc_scatter_x = _make_scatter(DX)


def _init_body(batchf, xp, zp, tb, tw1, tb1, tw2, tb2, cwz, cwt, cb,
               we1d, we1s, h_out, gd_out, gs_out):
    lane = lax.broadcasted_iota(jnp.int32, (VB, 128), 1).astype(F32)
    oh = (batchf[...] == lane).astype(F32)
    z_node = jnp.dot(oh, zp[...], preferred_element_type=F32)
    tn = jnp.dot(oh, tb[...], preferred_element_type=F32)
    temb = jnp.dot(_silu(tn * tw1[0:1, :] + tb1[0:1, :]), tw2[...],
                   preferred_element_type=F32) + tb2[0:1, :]
    h = (jnp.dot(z_node, cwz[...], preferred_element_type=F32)
         + jnp.dot(temb, cwt[...], preferred_element_type=F32) + cb[0:1, :])
    xv = xp[...]
    h_out[...] = h
    gd_out[:, :128] = jnp.dot(h, we1d[...], preferred_element_type=F32)
    gd_out[:, 128:] = xv
    gs_out[:, :128] = jnp.dot(h, we1s[...], preferred_element_type=F32)
    gs_out[:, 128:] = -xv


def _edge_body(last, r_ref, we2, be2, wx1, bx1, wx2r, bx2b, wd, be1, out_ref):
    rows = r_ref[...]
    diff = rows[:, 128:]
    dist2 = jnp.sum(diff * diff, axis=1, keepdims=True)
    pre = rows[:, :128] + dist2 * wd[0:1, :] + be1[0:1, :]
    u = _silu(pre)
    m = _silu(jnp.dot(u, we2[...], preferred_element_type=F32) + be2[0:1, :])
    c1 = _silu(jnp.dot(m, wx1[...], preferred_element_type=F32) + bx1[0:1, :])
    cw = jnp.sum(c1 * wx2r[0:1, :], axis=1, keepdims=True) + bx2b[0:1, 0:1]
    ones3 = (lax.broadcasted_iota(jnp.int32, (EB, DX), 1) == 3).astype(F32)
    xout = diff * cw + ones3
    if last:
        out_ref[...] = xout
    else:
        out_ref[:, :128] = m
        out_ref[:, 128:] = xout


def _node_body(p0, p1, h, xp, wh1h, wh1m, bh1, wh2, bh2, we1d, we1s,
               hn_out, xn_out, gd_out, gs_out):
    pa = p0[...]
    pb = p1[...]
    magg = pa[:, :128] + pb[:, :128]
    xs = pa[:, 128:] + pb[:, 128:]
    lane = lax.broadcasted_iota(jnp.int32, (VB, DX), 1)
    deg = jnp.sum(xs * (lane == 3).astype(F32), axis=1, keepdims=True)
    coef = 1.0 / jnp.maximum(deg, 1.0)
    xn = xp[...] + xs * coef * (lane < 3).astype(F32)
    hh = h[...]
    g = jnp.dot(hh, wh1h[...], preferred_element_type=F32) \
        + jnp.dot(magg, wh1m[...], preferred_element_type=F32) + bh1[0:1, :]
    hn = hh + jnp.dot(_silu(g), wh2[...], preferred_element_type=F32) + bh2[0:1, :]
    hn_out[...] = hn
    xn_out[...] = xn
    gd_out[:, :128] = jnp.dot(hn, we1d[...], preferred_element_type=F32)
    gd_out[:, 128:] = xn
    gs_out[:, :128] = jnp.dot(hn, we1s[...], preferred_element_type=F32)
    gs_out[:, 128:] = -xn


def _fin_body(p0, p1, xp, x0, vel_out):
    xs = p0[...] + p1[...]
    lane = lax.broadcasted_iota(jnp.int32, (VB, DX), 1)
    deg = jnp.sum(xs * (lane == 3).astype(F32), axis=1, keepdims=True)
    coef = 1.0 / jnp.maximum(deg, 1.0)
    vel_out[...] = xp[...] + xs * coef * (lane < 3).astype(F32) - x0[...]


def _wspec(shape):
    nd = len(shape)
    return pl.BlockSpec(shape, lambda i: (0,) * nd)


_init_call = pl.pallas_call(
    _init_body,
    grid=(VP // VB,),
    in_specs=[
        pl.BlockSpec((VB, 128), lambda i: (i, 0)),
        pl.BlockSpec((VB, DX), lambda i: (i, 0)),
        _wspec((128, LD)), _wspec((128, TD)),
        _wspec((8, TD)), _wspec((8, TD)), _wspec((TD, TD)), _wspec((8, TD)),
        _wspec((LD, HD)), _wspec((TD, HD)), _wspec((8, HD)),
        _wspec((HD, HD)), _wspec((HD, HD)),
    ],
    out_specs=[
        pl.BlockSpec((VB, HD), lambda i: (i, 0)),
        pl.BlockSpec((VB, DW), lambda i: (i, 0)),
        pl.BlockSpec((VB, DW), lambda i: (i, 0)),
    ],
    out_shape=[
        jax.ShapeDtypeStruct((VP, HD), F32),
        jax.ShapeDtypeStruct((VP, DW), F32),
        jax.ShapeDtypeStruct((VP, DW), F32),
    ],
)


def _make_edge(last):
    dwo = DX if last else DW
    return pl.pallas_call(
        functools.partial(_edge_body, last),
        grid=(EP // EB,),
        in_specs=[
            pl.BlockSpec((EB, DW), lambda i: (i, 0)),
            _wspec((HD, HD)), _wspec((8, HD)),
            _wspec((HD, HD)), _wspec((8, HD)),
            _wspec((8, HD)), _wspec((8, 8)),
            _wspec((8, HD)), _wspec((8, HD)),
        ],
        out_specs=pl.BlockSpec((EB, dwo), lambda i: (i, 0)),
        out_shape=jax.ShapeDtypeStruct((EP, dwo), F32),
    )


_edge_mid = _make_edge(False)
_edge_last = _make_edge(True)

_node_call = pl.pallas_call(
    _node_body,
    grid=(VP // VB,),
    in_specs=[
        pl.BlockSpec((VB, DW), lambda i: (i, 0)),
        pl.BlockSpec((VB, DW), lambda i: (i, 0)),
        pl.BlockSpec((VB, HD), lambda i: (i, 0)),
        pl.BlockSpec((VB, DX), lambda i: (i, 0)),
        _wspec((HD, HD)), _wspec((HD, HD)), _wspec((8, HD)),
        _wspec((HD, HD)), _wspec((8, HD)),
        _wspec((HD, HD)), _wspec((HD, HD)),
    ],
    out_specs=[
        pl.BlockSpec((VB, HD), lambda i: (i, 0)),
        pl.BlockSpec((VB, DX), lambda i: (i, 0)),
        pl.BlockSpec((VB, DW), lambda i: (i, 0)),
        pl.BlockSpec((VB, DW), lambda i: (i, 0)),
    ],
    out_shape=[
        jax.ShapeDtypeStruct((VP, HD), F32),
        jax.ShapeDtypeStruct((VP, DX), F32),
        jax.ShapeDtypeStruct((VP, DW), F32),
        jax.ShapeDtypeStruct((VP, DW), F32),
    ],
)

_fin_call = pl.pallas_call(
    _fin_body,
    grid=(VP // VB,),
    in_specs=[
        pl.BlockSpec((VB, DX), lambda i: (i, 0)),
        pl.BlockSpec((VB, DX), lambda i: (i, 0)),
        pl.BlockSpec((VB, DX), lambda i: (i, 0)),
        pl.BlockSpec((VB, DX), lambda i: (i, 0)),
    ],
    out_specs=pl.BlockSpec((VB, DX), lambda i: (i, 0)),
    out_shape=jax.ShapeDtypeStruct((VP, DX), F32),
)


def _bc8(v):
    v = v.reshape(-1)
    return jnp.broadcast_to(v[None, :], (8, v.shape[0]))


def kernel(pos, edge_index, batch, t, z, t_W1, t_b1, t_W2, t_b2, c_W, c_b,
           We1, be1, We2, be2, Wx1, bx1, Wx2, bx2, Wh1, bh1, Wh2, bh2,
           Wout, bout):
    src = edge_index[0].astype(jnp.int32)
    dst = edge_index[1].astype(jnp.int32)
    epad = jnp.full((EP - E,), V, jnp.int32)
    srcp = jnp.concatenate([src, epad])
    dstp = jnp.concatenate([dst, epad])

    xp0 = jnp.zeros((VP, DX), F32).at[:V, :3].set(pos)
    batchf = jnp.zeros((VP, 128), F32).at[:V, :].set(batch.astype(F32)[:, None])
    zp = jnp.zeros((128, LD), F32).at[:B].set(z)
    tb = jnp.zeros((128, TD), F32).at[:B].set(t[:, None])

    h, gd, gs = _init_call(
        batchf, xp0, zp, tb,
        _bc8(t_W1), _bc8(t_b1), t_W2, _bc8(t_b2),
        c_W[:LD], c_W[LD:], _bc8(c_b),
        We1[0, :HD], We1[0, HD:2 * HD],
    )

    xp = xp0
    velf = None
    for l in range(NL):
        r = _sc_gather(gd, gs, dstp, srcp)
        last = l == NL - 1
        edge_fn = _edge_last if last else _edge_mid
        w = edge_fn(
            r, We2[l], _bc8(be2[l]), Wx1[l], _bc8(bx1[l]),
            _bc8(Wx2[l]), jnp.broadcast_to(bx2[l].reshape(1, 1), (8, 8)),
            _bc8(We1[l, 2 * HD]), _bc8(be1[l]),
        )
        p = (_sc_scatter_x if last else _sc_scatter_full)(w, dstp)
        if last:
            velf = _fin_call(p[0], p[1], xp, xp0)
        else:
            h, xp, gd, gs = _node_call(
                p[0], p[1], h, xp,
                Wh1[l, :HD], Wh1[l, HD:], _bc8(bh1[l]), Wh2[l], _bc8(bh2[l]),
                We1[l + 1, :HD], We1[l + 1, HD:2 * HD],
            )
    return velf[:V, :3]


# speedup vs baseline: 1.9261x; 1.0070x over previous
import functools

import jax
import jax.numpy as jnp
from jax import lax
from jax.experimental import pallas as pl
from jax.experimental.pallas import tpu as pltpu
from jax.experimental.pallas import tpu_sc as plsc

V = 10000
E = 320000
B = 8
HD = 128
LD = 64
TD = 16
NL = 4

VP = 10240
EP = 327680
DW = 144
DX = 16
NW = 32
EPW = EP // NW
CH = 128
NCH = EPW // CH
RPT = VP // 16

EB = 2048
VB = 1024

F32 = jnp.float32


def _silu(x):
    return x * jax.nn.sigmoid(x)


_SC_MESH = plsc.VectorSubcoreMesh(
    core_axis_name="c", subcore_axis_name="s", num_cores=2, num_subcores=16)


def _sc_gather_body(gd, gs, dstp, srcp, out,
                    didx0, didx1, sidx0, sidx1,
                    bufa0, bufa1, bufb0, bufb1,
                    isem0, isem1, gsem0, gsem1, osem0, osem1):
    c = lax.axis_index("c")
    s = lax.axis_index("s")
    wid = s * 2 + c
    base_w = wid * EPW
    didx = (didx0, didx1)
    sidx = (sidx0, sidx1)
    bufa = (bufa0, bufa1)
    bufb = (bufb0, bufb1)
    isem = (isem0, isem1)
    gsem = (gsem0, gsem1)
    osem = (osem0, osem1)

    def start_idx(i, b):
        base = base_w + i * CH
        pltpu.async_copy(dstp.at[pl.ds(base, CH)], didx[b], isem[b])
        pltpu.async_copy(srcp.at[pl.ds(base, CH)], sidx[b], isem[b])

    def wait_idx(i, b):
        base = base_w + i * CH
        pltpu.make_async_copy(dstp.at[pl.ds(base, CH)], didx[b], isem[b]).wait()
        pltpu.make_async_copy(srcp.at[pl.ds(base, CH)], sidx[b], isem[b]).wait()

    start_idx(0, 0)

    def pair(p, carry):
        for b in range(2):
            i = 2 * p + b
            wait_idx(i, b)

            @pl.when(p >= 1)
            def _wait_out():
                pltpu.make_async_copy(
                    bufa[b], out.at[pl.ds(base_w + (i - 2) * CH, CH)],
                    osem[b]).wait()

            pltpu.async_copy(gd.at[didx[b]], bufa[b], gsem[b])
            pltpu.async_copy(gs.at[sidx[b]], bufb[b], gsem[b])
            if b == 0:
                start_idx(i + 1, 1)
            else:
                @pl.when(p < NCH // 2 - 1)
                def _nxt():
                    start_idx(i + 1, 0)
            pltpu.make_async_copy(gd.at[didx[b]], bufa[b], gsem[b]).wait()
            pltpu.make_async_copy(gs.at[sidx[b]], bufb[b], gsem[b]).wait()

            def addrow(r, cc):
                for j in range(DW // 16):
                    sl = pl.ds(j * 16, 16)
                    bufa[b][r, sl] = bufa[b][r, sl] + bufb[b][r, sl]
                return cc

            lax.fori_loop(0, CH, addrow, 0)
            pltpu.async_copy(bufa[b], out.at[pl.ds(base_w + i * CH, CH)],
                             osem[b])
        return carry

    lax.fori_loop(0, NCH // 2, pair, 0)
    for b in range(2):
        last = NCH - 2 + b
        pltpu.make_async_copy(
            bufa[b], out.at[pl.ds(base_w + last * CH, CH)], osem[b]).wait()


_SC_PARAMS = pltpu.CompilerParams(use_tc_tiling_on_sc=False)

_sc_gather = pl.kernel(
    _sc_gather_body,
    out_type=jax.ShapeDtypeStruct((EP, DW), F32),
    mesh=_SC_MESH,
    compiler_params=_SC_PARAMS,
    scratch_types=[
        pltpu.VMEM((CH,), jnp.int32),
        pltpu.VMEM((CH,), jnp.int32),
        pltpu.VMEM((CH,), jnp.int32),
        pltpu.VMEM((CH,), jnp.int32),
        pltpu.VMEM((CH, DW), F32),
        pltpu.VMEM((CH, DW), F32),
        pltpu.VMEM((CH, DW), F32),
        pltpu.VMEM((CH, DW), F32),
        pltpu.SemaphoreType.DMA,
        pltpu.SemaphoreType.DMA,
        pltpu.SemaphoreType.DMA,
        pltpu.SemaphoreType.DMA,
        pltpu.SemaphoreType.DMA,
        pltpu.SemaphoreType.DMA,
    ],
)


def _sc_scatter_body(dw, w_hbm, dstp, out, idxb, buf, zbuf, acc):
    c = lax.axis_index("c")
    s = lax.axis_index("s")
    wid = s * 2 + c
    base_w = wid * EPW

    def zrow(r, cc):
        for j in range(dw // 16):
            zbuf[r, pl.ds(j * 16, 16)] = jnp.zeros((16,), F32)
        return cc

    lax.fori_loop(0, CH, zrow, 0)
    for k in range(RPT // CH):
        pltpu.sync_copy(zbuf, acc.at[pl.ds(s * RPT + k * CH, CH)])
    plsc.subcore_barrier()

    def chunk(i, carry):
        base = base_w + i * CH
        pltpu.sync_copy(dstp.at[pl.ds(base, CH)], idxb)
        pltpu.sync_copy(w_hbm.at[pl.ds(base, CH)], buf)
        pltpu.sync_copy(buf, acc.at[idxb], add=True)
        return carry

    lax.fori_loop(0, NCH, chunk, 0)
    plsc.subcore_barrier()
    for k in range(RPT // CH):
        r0 = s * RPT + k * CH
        pltpu.sync_copy(acc.at[pl.ds(r0, CH)], buf)
        pltpu.sync_copy(buf, out.at[c, pl.ds(r0, CH)])


def _make_scatter(dw):
    return pl.kernel(
        functools.partial(_sc_scatter_body, dw),
        out_type=jax.ShapeDtypeStruct((2, VP, dw), F32),
        mesh=_SC_MESH,
        compiler_params=_SC_PARAMS,
        scratch_types=[
            pltpu.VMEM((CH,), jnp.int32),
            pltpu.VMEM((CH, dw), F32),
            pltpu.VMEM((CH, dw), F32),
            pltpu.VMEM_SHARED((VP, dw), F32),
        ],
    )


_sc_scatter_full = _make_scatter(DW)
_sc_scatter_x = _make_scatter(DX)


def _init_body(batchf, xp, zp, tb, tw1, tb1, tw2, tb2, cwz, cwt, cb,
               we1d, we1s, h_out, gd_out, gs_out):
    lane = lax.broadcasted_iota(jnp.int32, (VB, 128), 1).astype(F32)
    oh = (batchf[...] == lane).astype(F32)
    z_node = jnp.dot(oh, zp[...], preferred_element_type=F32)
    tn = jnp.dot(oh, tb[...], preferred_element_type=F32)
    temb = jnp.dot(_silu(tn * tw1[0:1, :] + tb1[0:1, :]), tw2[...],
                   preferred_element_type=F32) + tb2[0:1, :]
    h = (jnp.dot(z_node, cwz[...], preferred_element_type=F32)
         + jnp.dot(temb, cwt[...], preferred_element_type=F32) + cb[0:1, :])
    xv = xp[...]
    h_out[...] = h
    gd_out[:, :128] = jnp.dot(h, we1d[...], preferred_element_type=F32)
    gd_out[:, 128:] = xv
    gs_out[:, :128] = jnp.dot(h, we1s[...], preferred_element_type=F32)
    gs_out[:, 128:] = -xv


def _edge_body(last, r_ref, we2, be2, wx1, bx1, wx2r, bx2b, wd, be1, out_ref):
    rows = r_ref[...]
    diff = rows[:, 128:]
    dist2 = jnp.sum(diff * diff, axis=1, keepdims=True)
    pre = rows[:, :128] + dist2 * wd[0:1, :] + be1[0:1, :]
    u = _silu(pre)
    m = _silu(jnp.dot(u, we2[...], preferred_element_type=F32) + be2[0:1, :])
    c1 = _silu(jnp.dot(m, wx1[...], preferred_element_type=F32) + bx1[0:1, :])
    cw = jnp.sum(c1 * wx2r[0:1, :], axis=1, keepdims=True) + bx2b[0:1, 0:1]
    ones3 = (lax.broadcasted_iota(jnp.int32, (EB, DX), 1) == 3).astype(F32)
    xout = diff * cw + ones3
    if last:
        out_ref[...] = xout
    else:
        out_ref[:, :128] = m
        out_ref[:, 128:] = xout


def _node_body(p0, p1, h, xp, wh1h, wh1m, bh1, wh2, bh2, we1d, we1s,
               hn_out, xn_out, gd_out, gs_out):
    pa = p0[...]
    pb = p1[...]
    magg = pa[:, :128] + pb[:, :128]
    xs = pa[:, 128:] + pb[:, 128:]
    lane = lax.broadcasted_iota(jnp.int32, (VB, DX), 1)
    deg = jnp.sum(xs * (lane == 3).astype(F32), axis=1, keepdims=True)
    coef = 1.0 / jnp.maximum(deg, 1.0)
    xn = xp[...] + xs * coef * (lane < 3).astype(F32)
    hh = h[...]
    g = jnp.dot(hh, wh1h[...], preferred_element_type=F32) \
        + jnp.dot(magg, wh1m[...], preferred_element_type=F32) + bh1[0:1, :]
    hn = hh + jnp.dot(_silu(g), wh2[...], preferred_element_type=F32) + bh2[0:1, :]
    hn_out[...] = hn
    xn_out[...] = xn
    gd_out[:, :128] = jnp.dot(hn, we1d[...], preferred_element_type=F32)
    gd_out[:, 128:] = xn
    gs_out[:, :128] = jnp.dot(hn, we1s[...], preferred_element_type=F32)
    gs_out[:, 128:] = -xn


def _fin_body(p0, p1, xp, x0, vel_out):
    xs = p0[...] + p1[...]
    lane = lax.broadcasted_iota(jnp.int32, (VB, DX), 1)
    deg = jnp.sum(xs * (lane == 3).astype(F32), axis=1, keepdims=True)
    coef = 1.0 / jnp.maximum(deg, 1.0)
    vel_out[...] = xp[...] + xs * coef * (lane < 3).astype(F32) - x0[...]


def _wspec(shape):
    nd = len(shape)
    return pl.BlockSpec(shape, lambda i: (0,) * nd)


_init_call = pl.pallas_call(
    _init_body,
    grid=(VP // VB,),
    in_specs=[
        pl.BlockSpec((VB, 128), lambda i: (i, 0)),
        pl.BlockSpec((VB, DX), lambda i: (i, 0)),
        _wspec((128, LD)), _wspec((128, TD)),
        _wspec((8, TD)), _wspec((8, TD)), _wspec((TD, TD)), _wspec((8, TD)),
        _wspec((LD, HD)), _wspec((TD, HD)), _wspec((8, HD)),
        _wspec((HD, HD)), _wspec((HD, HD)),
    ],
    out_specs=[
        pl.BlockSpec((VB, HD), lambda i: (i, 0)),
        pl.BlockSpec((VB, DW), lambda i: (i, 0)),
        pl.BlockSpec((VB, DW), lambda i: (i, 0)),
    ],
    out_shape=[
        jax.ShapeDtypeStruct((VP, HD), F32),
        jax.ShapeDtypeStruct((VP, DW), F32),
        jax.ShapeDtypeStruct((VP, DW), F32),
    ],
)


def _make_edge(last):
    dwo = DX if last else DW
    return pl.pallas_call(
        functools.partial(_edge_body, last),
        grid=(EP // EB,),
        in_specs=[
            pl.BlockSpec((EB, DW), lambda i: (i, 0)),
            _wspec((HD, HD)), _wspec((8, HD)),
            _wspec((HD, HD)), _wspec((8, HD)),
            _wspec((8, HD)), _wspec((8, 8)),
            _wspec((8, HD)), _wspec((8, HD)),
        ],
        out_specs=pl.BlockSpec((EB, dwo), lambda i: (i, 0)),
        out_shape=jax.ShapeDtypeStruct((EP, dwo), F32),
    )


_edge_mid = _make_edge(False)
_edge_last = _make_edge(True)

_node_call = pl.pallas_call(
    _node_body,
    grid=(VP // VB,),
    in_specs=[
        pl.BlockSpec((VB, DW), lambda i: (i, 0)),
        pl.BlockSpec((VB, DW), lambda i: (i, 0)),
        pl.BlockSpec((VB, HD), lambda i: (i, 0)),
        pl.BlockSpec((VB, DX), lambda i: (i, 0)),
        _wspec((HD, HD)), _wspec((HD, HD)), _wspec((8, HD)),
        _wspec((HD, HD)), _wspec((8, HD)),
        _wspec((HD, HD)), _wspec((HD, HD)),
    ],
    out_specs=[
        pl.BlockSpec((VB, HD), lambda i: (i, 0)),
        pl.BlockSpec((VB, DX), lambda i: (i, 0)),
        pl.BlockSpec((VB, DW), lambda i: (i, 0)),
        pl.BlockSpec((VB, DW), lambda i: (i, 0)),
    ],
    out_shape=[
        jax.ShapeDtypeStruct((VP, HD), F32),
        jax.ShapeDtypeStruct((VP, DX), F32),
        jax.ShapeDtypeStruct((VP, DW), F32),
        jax.ShapeDtypeStruct((VP, DW), F32),
    ],
)

_fin_call = pl.pallas_call(
    _fin_body,
    grid=(VP // VB,),
    in_specs=[
        pl.BlockSpec((VB, DX), lambda i: (i, 0)),
        pl.BlockSpec((VB, DX), lambda i: (i, 0)),
        pl.BlockSpec((VB, DX), lambda i: (i, 0)),
        pl.BlockSpec((VB, DX), lambda i: (i, 0)),
    ],
    out_specs=pl.BlockSpec((VB, DX), lambda i: (i, 0)),
    out_shape=jax.ShapeDtypeStruct((VP, DX), F32),
)


def _bc8(v):
    v = v.reshape(-1)
    return jnp.broadcast_to(v[None, :], (8, v.shape[0]))


def kernel(pos, edge_index, batch, t, z, t_W1, t_b1, t_W2, t_b2, c_W, c_b,
           We1, be1, We2, be2, Wx1, bx1, Wx2, bx2, Wh1, bh1, Wh2, bh2,
           Wout, bout):
    src = edge_index[0].astype(jnp.int32)
    dst = edge_index[1].astype(jnp.int32)
    epad = jnp.full((EP - E,), V, jnp.int32)
    srcp = jnp.concatenate([src, epad])
    dstp = jnp.concatenate([dst, epad])

    xp0 = jnp.zeros((VP, DX), F32).at[:V, :3].set(pos)
    batchf = jnp.zeros((VP, 128), F32).at[:V, :].set(batch.astype(F32)[:, None])
    zp = jnp.zeros((128, LD), F32).at[:B].set(z)
    tb = jnp.zeros((128, TD), F32).at[:B].set(t[:, None])

    h, gd, gs = _init_call(
        batchf, xp0, zp, tb,
        _bc8(t_W1), _bc8(t_b1), t_W2, _bc8(t_b2),
        c_W[:LD], c_W[LD:], _bc8(c_b),
        We1[0, :HD], We1[0, HD:2 * HD],
    )

    xp = xp0
    velf = None
    for l in range(NL):
        r = _sc_gather(gd, gs, dstp, srcp)
        last = l == NL - 1
        edge_fn = _edge_last if last else _edge_mid
        w = edge_fn(
            r, We2[l], _bc8(be2[l]), Wx1[l], _bc8(bx1[l]),
            _bc8(Wx2[l]), jnp.broadcast_to(bx2[l].reshape(1, 1), (8, 8)),
            _bc8(We1[l, 2 * HD]), _bc8(be1[l]),
        )
        p = (_sc_scatter_x if last else _sc_scatter_full)(w, dstp)
        if last:
            velf = _fin_call(p[0], p[1], xp, xp0)
        else:
            h, xp, gd, gs = _node_call(
                p[0], p[1], h, xp,
                Wh1[l, :HD], Wh1[l, HD:], _bc8(bh1[l]), Wh2[l], _bc8(bh2[l]),
                We1[l + 1, :HD], We1[l + 1, HD:2 * HD],
            )
    return velf[:V, :3]


# speedup vs baseline: 2.1570x; 1.1199x over previous
import functools

import jax
import jax.numpy as jnp
from jax import lax
from jax.experimental import pallas as pl
from jax.experimental.pallas import tpu as pltpu
from jax.experimental.pallas import tpu_sc as plsc

V = 10000
E = 320000
B = 8
HD = 128
LD = 64
TD = 16
NL = 4

VP = 10240
EP = 327680
DW = 144
DX = 16
NW = 32
EPW = EP // NW
CH = 128
NCH = EPW // CH
RPT = VP // 16

EB = 2048
VB = 1024

F32 = jnp.float32


def _silu(x):
    return x * jax.nn.sigmoid(x)


_SC_MESH = plsc.VectorSubcoreMesh(
    core_axis_name="c", subcore_axis_name="s", num_cores=2, num_subcores=16)


def _sc_gather_body(gd, gs, dstp, srcp, out,
                    didx0, didx1, sidx0, sidx1,
                    bufa0, bufa1, bufb0, bufb1,
                    isem0, isem1, gsem0, gsem1, osem0, osem1):
    c = lax.axis_index("c")
    s = lax.axis_index("s")
    wid = s * 2 + c
    base_w = wid * EPW
    didx = (didx0, didx1)
    sidx = (sidx0, sidx1)
    bufa = (bufa0, bufa1)
    bufb = (bufb0, bufb1)
    isem = (isem0, isem1)
    gsem = (gsem0, gsem1)
    osem = (osem0, osem1)

    def start_idx(i, b):
        base = base_w + i * CH
        pltpu.async_copy(dstp.at[pl.ds(base, CH)], didx[b], isem[b])
        pltpu.async_copy(srcp.at[pl.ds(base, CH)], sidx[b], isem[b])

    def wait_idx(i, b):
        base = base_w + i * CH
        pltpu.make_async_copy(dstp.at[pl.ds(base, CH)], didx[b], isem[b]).wait()
        pltpu.make_async_copy(srcp.at[pl.ds(base, CH)], sidx[b], isem[b]).wait()

    def start_gather(b):
        pltpu.async_copy(gd.at[didx[b]], bufa[b], gsem[b])
        pltpu.async_copy(gs.at[sidx[b]], bufb[b], gsem[b])

    def wait_gather(b):
        pltpu.make_async_copy(gd.at[didx[b]], bufa[b], gsem[b]).wait()
        pltpu.make_async_copy(gs.at[sidx[b]], bufb[b], gsem[b]).wait()

    def wait_out(i, b):
        pltpu.make_async_copy(
            bufa[b], out.at[pl.ds(base_w + i * CH, CH)], osem[b]).wait()

    start_idx(0, 0)
    wait_idx(0, 0)
    start_gather(0)
    start_idx(1, 1)

    def pair(p, carry):
        for b in range(2):
            i = 2 * p + b
            nb = 1 - b

            def _launch_next():
                wait_idx(i + 1, nb)
                if b == 0:
                    @pl.when(p >= 1)
                    def _wo():
                        wait_out(i - 1, nb)
                else:
                    wait_out(i - 1, nb)
                start_gather(nb)

            if b == 0:
                _launch_next()
            else:
                @pl.when(p < NCH // 2 - 1)
                def _ln():
                    _launch_next()

            wait_gather(b)

            @pl.when(p < NCH // 2 - 1)
            def _nidx():
                start_idx(i + 2, b)

            @plsc.parallel_loop(0, CH, 1, unroll=8)
            def _add(r):
                for j in range(DW // 16):
                    sl = pl.ds(j * 16, 16)
                    bufa[b][r, sl] = bufa[b][r, sl] + bufb[b][r, sl]

            pltpu.async_copy(bufa[b], out.at[pl.ds(base_w + i * CH, CH)],
                             osem[b])
        return carry

    lax.fori_loop(0, NCH // 2, pair, 0)
    for b in range(2):
        wait_out(NCH - 2 + b, b)


_SC_PARAMS = pltpu.CompilerParams(use_tc_tiling_on_sc=False)

_sc_gather = pl.kernel(
    _sc_gather_body,
    out_type=jax.ShapeDtypeStruct((EP, DW), F32),
    mesh=_SC_MESH,
    compiler_params=_SC_PARAMS,
    scratch_types=[
        pltpu.VMEM((CH,), jnp.int32),
        pltpu.VMEM((CH,), jnp.int32),
        pltpu.VMEM((CH,), jnp.int32),
        pltpu.VMEM((CH,), jnp.int32),
        pltpu.VMEM((CH, DW), F32),
        pltpu.VMEM((CH, DW), F32),
        pltpu.VMEM((CH, DW), F32),
        pltpu.VMEM((CH, DW), F32),
        pltpu.SemaphoreType.DMA,
        pltpu.SemaphoreType.DMA,
        pltpu.SemaphoreType.DMA,
        pltpu.SemaphoreType.DMA,
        pltpu.SemaphoreType.DMA,
        pltpu.SemaphoreType.DMA,
    ],
)


def _sc_scatter_body(dw, w_hbm, dstp, out, idxb, buf, zbuf, acc):
    c = lax.axis_index("c")
    s = lax.axis_index("s")
    wid = s * 2 + c
    base_w = wid * EPW

    def zrow(r, cc):
        for j in range(dw // 16):
            zbuf[r, pl.ds(j * 16, 16)] = jnp.zeros((16,), F32)
        return cc

    lax.fori_loop(0, CH, zrow, 0)
    for k in range(RPT // CH):
        pltpu.sync_copy(zbuf, acc.at[pl.ds(s * RPT + k * CH, CH)])
    plsc.subcore_barrier()

    def chunk(i, carry):
        base = base_w + i * CH
        pltpu.sync_copy(dstp.at[pl.ds(base, CH)], idxb)
        pltpu.sync_copy(w_hbm.at[pl.ds(base, CH)], buf)
        pltpu.sync_copy(buf, acc.at[idxb], add=True)
        return carry

    lax.fori_loop(0, NCH, chunk, 0)
    plsc.subcore_barrier()
    for k in range(RPT // CH):
        r0 = s * RPT + k * CH
        pltpu.sync_copy(acc.at[pl.ds(r0, CH)], buf)
        pltpu.sync_copy(buf, out.at[c, pl.ds(r0, CH)])


def _make_scatter(dw):
    return pl.kernel(
        functools.partial(_sc_scatter_body, dw),
        out_type=jax.ShapeDtypeStruct((2, VP, dw), F32),
        mesh=_SC_MESH,
        compiler_params=_SC_PARAMS,
        scratch_types=[
            pltpu.VMEM((CH,), jnp.int32),
            pltpu.VMEM((CH, dw), F32),
            pltpu.VMEM((CH, dw), F32),
            pltpu.VMEM_SHARED((VP, dw), F32),
        ],
    )


_sc_scatter_full = _make_scatter(DW)
_sc_scatter_x = _make_scatter(DX)


def _init_body(batchf, xp, zp, tb, tw1, tb1, tw2, tb2, cwz, cwt, cb,
               we1d, we1s, h_out, gd_out, gs_out):
    lane = lax.broadcasted_iota(jnp.int32, (VB, 128), 1).astype(F32)
    oh = (batchf[...] == lane).astype(F32)
    z_node = jnp.dot(oh, zp[...], preferred_element_type=F32)
    tn = jnp.dot(oh, tb[...], preferred_element_type=F32)
    temb = jnp.dot(_silu(tn * tw1[0:1, :] + tb1[0:1, :]), tw2[...],
                   preferred_element_type=F32) + tb2[0:1, :]
    h = (jnp.dot(z_node, cwz[...], preferred_element_type=F32)
         + jnp.dot(temb, cwt[...], preferred_element_type=F32) + cb[0:1, :])
    xv = xp[...]
    h_out[...] = h
    gd_out[:, :128] = jnp.dot(h, we1d[...], preferred_element_type=F32)
    gd_out[:, 128:] = xv
    gs_out[:, :128] = jnp.dot(h, we1s[...], preferred_element_type=F32)
    gs_out[:, 128:] = -xv


def _edge_body(last, r_ref, we2, be2, wx1, bx1, wx2r, bx2b, wd, be1, out_ref):
    rows = r_ref[...]
    diff = rows[:, 128:]
    dist2 = jnp.sum(diff * diff, axis=1, keepdims=True)
    pre = rows[:, :128] + dist2 * wd[0:1, :] + be1[0:1, :]
    u = _silu(pre)
    m = _silu(jnp.dot(u, we2[...], preferred_element_type=F32) + be2[0:1, :])
    c1 = _silu(jnp.dot(m, wx1[...], preferred_element_type=F32) + bx1[0:1, :])
    cw = jnp.sum(c1 * wx2r[0:1, :], axis=1, keepdims=True) + bx2b[0:1, 0:1]
    ones3 = (lax.broadcasted_iota(jnp.int32, (EB, DX), 1) == 3).astype(F32)
    xout = diff * cw + ones3
    if last:
        out_ref[...] = xout
    else:
        out_ref[:, :128] = m
        out_ref[:, 128:] = xout


def _node_body(p0, p1, h, xp, wh1h, wh1m, bh1, wh2, bh2, we1d, we1s,
               hn_out, xn_out, gd_out, gs_out):
    pa = p0[...]
    pb = p1[...]
    magg = pa[:, :128] + pb[:, :128]
    xs = pa[:, 128:] + pb[:, 128:]
    lane = lax.broadcasted_iota(jnp.int32, (VB, DX), 1)
    deg = jnp.sum(xs * (lane == 3).astype(F32), axis=1, keepdims=True)
    coef = 1.0 / jnp.maximum(deg, 1.0)
    xn = xp[...] + xs * coef * (lane < 3).astype(F32)
    hh = h[...]
    g = jnp.dot(hh, wh1h[...], preferred_element_type=F32) \
        + jnp.dot(magg, wh1m[...], preferred_element_type=F32) + bh1[0:1, :]
    hn = hh + jnp.dot(_silu(g), wh2[...], preferred_element_type=F32) + bh2[0:1, :]
    hn_out[...] = hn
    xn_out[...] = xn
    gd_out[:, :128] = jnp.dot(hn, we1d[...], preferred_element_type=F32)
    gd_out[:, 128:] = xn
    gs_out[:, :128] = jnp.dot(hn, we1s[...], preferred_element_type=F32)
    gs_out[:, 128:] = -xn


def _fin_body(p0, p1, xp, x0, vel_out):
    xs = p0[...] + p1[...]
    lane = lax.broadcasted_iota(jnp.int32, (VB, DX), 1)
    deg = jnp.sum(xs * (lane == 3).astype(F32), axis=1, keepdims=True)
    coef = 1.0 / jnp.maximum(deg, 1.0)
    vel_out[...] = xp[...] + xs * coef * (lane < 3).astype(F32) - x0[...]


def _wspec(shape):
    nd = len(shape)
    return pl.BlockSpec(shape, lambda i: (0,) * nd)


_init_call = pl.pallas_call(
    _init_body,
    grid=(VP // VB,),
    in_specs=[
        pl.BlockSpec((VB, 128), lambda i: (i, 0)),
        pl.BlockSpec((VB, DX), lambda i: (i, 0)),
        _wspec((128, LD)), _wspec((128, TD)),
        _wspec((8, TD)), _wspec((8, TD)), _wspec((TD, TD)), _wspec((8, TD)),
        _wspec((LD, HD)), _wspec((TD, HD)), _wspec((8, HD)),
        _wspec((HD, HD)), _wspec((HD, HD)),
    ],
    out_specs=[
        pl.BlockSpec((VB, HD), lambda i: (i, 0)),
        pl.BlockSpec((VB, DW), lambda i: (i, 0)),
        pl.BlockSpec((VB, DW), lambda i: (i, 0)),
    ],
    out_shape=[
        jax.ShapeDtypeStruct((VP, HD), F32),
        jax.ShapeDtypeStruct((VP, DW), F32),
        jax.ShapeDtypeStruct((VP, DW), F32),
    ],
)


def _make_edge(last):
    dwo = DX if last else DW
    return pl.pallas_call(
        functools.partial(_edge_body, last),
        grid=(EP // EB,),
        in_specs=[
            pl.BlockSpec((EB, DW), lambda i: (i, 0)),
            _wspec((HD, HD)), _wspec((8, HD)),
            _wspec((HD, HD)), _wspec((8, HD)),
            _wspec((8, HD)), _wspec((8, 8)),
            _wspec((8, HD)), _wspec((8, HD)),
        ],
        out_specs=pl.BlockSpec((EB, dwo), lambda i: (i, 0)),
        out_shape=jax.ShapeDtypeStruct((EP, dwo), F32),
    )


_edge_mid = _make_edge(False)
_edge_last = _make_edge(True)

_node_call = pl.pallas_call(
    _node_body,
    grid=(VP // VB,),
    in_specs=[
        pl.BlockSpec((VB, DW), lambda i: (i, 0)),
        pl.BlockSpec((VB, DW), lambda i: (i, 0)),
        pl.BlockSpec((VB, HD), lambda i: (i, 0)),
        pl.BlockSpec((VB, DX), lambda i: (i, 0)),
        _wspec((HD, HD)), _wspec((HD, HD)), _wspec((8, HD)),
        _wspec((HD, HD)), _wspec((8, HD)),
        _wspec((HD, HD)), _wspec((HD, HD)),
    ],
    out_specs=[
        pl.BlockSpec((VB, HD), lambda i: (i, 0)),
        pl.BlockSpec((VB, DX), lambda i: (i, 0)),
        pl.BlockSpec((VB, DW), lambda i: (i, 0)),
        pl.BlockSpec((VB, DW), lambda i: (i, 0)),
    ],
    out_shape=[
        jax.ShapeDtypeStruct((VP, HD), F32),
        jax.ShapeDtypeStruct((VP, DX), F32),
        jax.ShapeDtypeStruct((VP, DW), F32),
        jax.ShapeDtypeStruct((VP, DW), F32),
    ],
)

_fin_call = pl.pallas_call(
    _fin_body,
    grid=(VP // VB,),
    in_specs=[
        pl.BlockSpec((VB, DX), lambda i: (i, 0)),
        pl.BlockSpec((VB, DX), lambda i: (i, 0)),
        pl.BlockSpec((VB, DX), lambda i: (i, 0)),
        pl.BlockSpec((VB, DX), lambda i: (i, 0)),
    ],
    out_specs=pl.BlockSpec((VB, DX), lambda i: (i, 0)),
    out_shape=jax.ShapeDtypeStruct((VP, DX), F32),
)


def _bc8(v):
    v = v.reshape(-1)
    return jnp.broadcast_to(v[None, :], (8, v.shape[0]))


def kernel(pos, edge_index, batch, t, z, t_W1, t_b1, t_W2, t_b2, c_W, c_b,
           We1, be1, We2, be2, Wx1, bx1, Wx2, bx2, Wh1, bh1, Wh2, bh2,
           Wout, bout):
    src = edge_index[0].astype(jnp.int32)
    dst = edge_index[1].astype(jnp.int32)
    epad = jnp.full((EP - E,), V, jnp.int32)
    srcp = jnp.concatenate([src, epad])
    dstp = jnp.concatenate([dst, epad])

    xp0 = jnp.zeros((VP, DX), F32).at[:V, :3].set(pos)
    batchf = jnp.zeros((VP, 128), F32).at[:V, :].set(batch.astype(F32)[:, None])
    zp = jnp.zeros((128, LD), F32).at[:B].set(z)
    tb = jnp.zeros((128, TD), F32).at[:B].set(t[:, None])

    h, gd, gs = _init_call(
        batchf, xp0, zp, tb,
        _bc8(t_W1), _bc8(t_b1), t_W2, _bc8(t_b2),
        c_W[:LD], c_W[LD:], _bc8(c_b),
        We1[0, :HD], We1[0, HD:2 * HD],
    )

    xp = xp0
    velf = None
    for l in range(NL):
        r = _sc_gather(gd, gs, dstp, srcp)
        last = l == NL - 1
        edge_fn = _edge_last if last else _edge_mid
        w = edge_fn(
            r, We2[l], _bc8(be2[l]), Wx1[l], _bc8(bx1[l]),
            _bc8(Wx2[l]), jnp.broadcast_to(bx2[l].reshape(1, 1), (8, 8)),
            _bc8(We1[l, 2 * HD]), _bc8(be1[l]),
        )
        p = (_sc_scatter_x if last else _sc_scatter_full)(w, dstp)
        if last:
            velf = _fin_call(p[0], p[1], xp, xp0)
        else:
            h, xp, gd, gs = _node_call(
                p[0], p[1], h, xp,
                Wh1[l, :HD], Wh1[l, HD:], _bc8(bh1[l]), Wh2[l], _bc8(bh2[l]),
                We1[l + 1, :HD], We1[l + 1, HD:2 * HD],
            )
    return velf[:V, :3]


# speedup vs baseline: 2.2491x; 1.0427x over previous
import functools

import jax
import jax.numpy as jnp
from jax import lax
from jax.experimental import pallas as pl
from jax.experimental.pallas import tpu as pltpu
from jax.experimental.pallas import tpu_sc as plsc

V = 10000
E = 320000
B = 8
HD = 128
LD = 64
TD = 16
NL = 4

VP = 10240
EP = 327680
DW = 144
DX = 16
NW = 32
EPW = EP // NW
CH = 128
NCH = EPW // CH
RPT = VP // 16

EB = 2048
VB = 1024

F32 = jnp.float32


def _silu(x):
    return x * jax.nn.sigmoid(x)


_SC_MESH = plsc.VectorSubcoreMesh(
    core_axis_name="c", subcore_axis_name="s", num_cores=2, num_subcores=16)


def _sc_gather_body(gd, gs, dstp, srcp, out,
                    didx0, didx1, sidx0, sidx1,
                    bufa0, bufa1, bufb0, bufb1,
                    isem0, isem1, gsem0, gsem1, osem0, osem1):
    c = lax.axis_index("c")
    s = lax.axis_index("s")
    wid = s * 2 + c
    base_w = wid * EPW
    didx = (didx0, didx1)
    sidx = (sidx0, sidx1)
    bufa = (bufa0, bufa1)
    bufb = (bufb0, bufb1)
    isem = (isem0, isem1)
    gsem = (gsem0, gsem1)
    osem = (osem0, osem1)

    def start_idx(i, b):
        base = base_w + i * CH
        pltpu.async_copy(dstp.at[pl.ds(base, CH)], didx[b], isem[b])
        pltpu.async_copy(srcp.at[pl.ds(base, CH)], sidx[b], isem[b])

    def wait_idx(i, b):
        base = base_w + i * CH
        pltpu.make_async_copy(dstp.at[pl.ds(base, CH)], didx[b], isem[b]).wait()
        pltpu.make_async_copy(srcp.at[pl.ds(base, CH)], sidx[b], isem[b]).wait()

    def start_gather(b):
        pltpu.async_copy(gd.at[didx[b]], bufa[b], gsem[b])
        pltpu.async_copy(gs.at[sidx[b]], bufb[b], gsem[b])

    def wait_gather(b):
        pltpu.make_async_copy(gd.at[didx[b]], bufa[b], gsem[b]).wait()
        pltpu.make_async_copy(gs.at[sidx[b]], bufb[b], gsem[b]).wait()

    def wait_out(i, b):
        pltpu.make_async_copy(
            bufa[b], out.at[pl.ds(base_w + i * CH, CH)], osem[b]).wait()

    start_idx(0, 0)
    wait_idx(0, 0)
    start_gather(0)
    start_idx(1, 1)

    def pair(p, carry):
        for b in range(2):
            i = 2 * p + b
            nb = 1 - b

            def _launch_next():
                wait_idx(i + 1, nb)
                if b == 0:
                    @pl.when(p >= 1)
                    def _wo():
                        wait_out(i - 1, nb)
                else:
                    wait_out(i - 1, nb)
                start_gather(nb)

            if b == 0:
                _launch_next()
            else:
                @pl.when(p < NCH // 2 - 1)
                def _ln():
                    _launch_next()

            wait_gather(b)

            @pl.when(p < NCH // 2 - 1)
            def _nidx():
                start_idx(i + 2, b)

            @plsc.parallel_loop(0, CH, 1, unroll=8)
            def _add(r):
                for j in range(DW // 16):
                    sl = pl.ds(j * 16, 16)
                    bufa[b][r, sl] = bufa[b][r, sl] + bufb[b][r, sl]

            pltpu.async_copy(bufa[b], out.at[pl.ds(base_w + i * CH, CH)],
                             osem[b])
        return carry

    lax.fori_loop(0, NCH // 2, pair, 0)
    for b in range(2):
        wait_out(NCH - 2 + b, b)


_SC_PARAMS = pltpu.CompilerParams(use_tc_tiling_on_sc=False)

_sc_gather = pl.kernel(
    _sc_gather_body,
    out_type=jax.ShapeDtypeStruct((EP, DW), F32),
    mesh=_SC_MESH,
    compiler_params=_SC_PARAMS,
    scratch_types=[
        pltpu.VMEM((CH,), jnp.int32),
        pltpu.VMEM((CH,), jnp.int32),
        pltpu.VMEM((CH,), jnp.int32),
        pltpu.VMEM((CH,), jnp.int32),
        pltpu.VMEM((CH, DW), F32),
        pltpu.VMEM((CH, DW), F32),
        pltpu.VMEM((CH, DW), F32),
        pltpu.VMEM((CH, DW), F32),
        pltpu.SemaphoreType.DMA,
        pltpu.SemaphoreType.DMA,
        pltpu.SemaphoreType.DMA,
        pltpu.SemaphoreType.DMA,
        pltpu.SemaphoreType.DMA,
        pltpu.SemaphoreType.DMA,
    ],
)


def _sc_scatter_body(dw, w_hbm, dstp, out, idx0, idx1, buf0, buf1, acc,
                     lsem0, lsem1, ssem0, ssem1):
    c = lax.axis_index("c")
    s = lax.axis_index("s")
    wid = s * 2 + c
    base_w = wid * EPW
    idxb = (idx0, idx1)
    buf = (buf0, buf1)
    lsem = (lsem0, lsem1)
    ssem = (ssem0, ssem1)

    def zrow(r, cc):
        for j in range(dw // 16):
            buf0[r, pl.ds(j * 16, 16)] = jnp.zeros((16,), F32)
        return cc

    lax.fori_loop(0, CH, zrow, 0)
    for k in range(RPT // CH):
        pltpu.sync_copy(buf0, acc.at[pl.ds(s * RPT + k * CH, CH)])
    plsc.subcore_barrier()

    def start_load(i, b):
        base = base_w + i * CH
        pltpu.async_copy(dstp.at[pl.ds(base, CH)], idxb[b], lsem[b])
        pltpu.async_copy(w_hbm.at[pl.ds(base, CH)], buf[b], lsem[b])

    def wait_load(i, b):
        base = base_w + i * CH
        pltpu.make_async_copy(dstp.at[pl.ds(base, CH)], idxb[b], lsem[b]).wait()
        pltpu.make_async_copy(w_hbm.at[pl.ds(base, CH)], buf[b], lsem[b]).wait()

    def wait_scat(b):
        pltpu.make_async_copy(buf[b], acc.at[idxb[b]], ssem[b]).wait()

    start_load(0, 0)

    def pair(p, carry):
        for b in range(2):
            i = 2 * p + b
            nb = 1 - b
            wait_load(i, b)

            def _next_load():
                wait_scat(nb)
                start_load(i + 1, nb)

            if b == 0:
                @pl.when(p >= 1)
                def _nl0():
                    _next_load()

                @pl.when(p == 0)
                def _nl1():
                    start_load(i + 1, nb)
            else:
                @pl.when(p < NCH // 2 - 1)
                def _nl2():
                    _next_load()

            pltpu.async_copy(buf[b], acc.at[idxb[b]], ssem[b], add=True)
        return carry

    lax.fori_loop(0, NCH // 2, pair, 0)
    wait_scat(0)
    wait_scat(1)
    plsc.subcore_barrier()
    for k in range(RPT // CH):
        r0 = s * RPT + k * CH
        pltpu.sync_copy(acc.at[pl.ds(r0, CH)], buf0)
        pltpu.sync_copy(buf0, out.at[c, pl.ds(r0, CH)])


def _make_scatter(dw):
    return pl.kernel(
        functools.partial(_sc_scatter_body, dw),
        out_type=jax.ShapeDtypeStruct((2, VP, dw), F32),
        mesh=_SC_MESH,
        compiler_params=_SC_PARAMS,
        scratch_types=[
            pltpu.VMEM((CH,), jnp.int32),
            pltpu.VMEM((CH,), jnp.int32),
            pltpu.VMEM((CH, dw), F32),
            pltpu.VMEM((CH, dw), F32),
            pltpu.VMEM_SHARED((VP, dw), F32),
            pltpu.SemaphoreType.DMA,
            pltpu.SemaphoreType.DMA,
            pltpu.SemaphoreType.DMA,
            pltpu.SemaphoreType.DMA,
        ],
    )


_sc_scatter_full = _make_scatter(DW)
_sc_scatter_x = _make_scatter(DX)


def _init_body(batchf, xp, zp, tb, tw1, tb1, tw2, tb2, cwz, cwt, cb,
               we1d, we1s, h_out, gd_out, gs_out):
    lane = lax.broadcasted_iota(jnp.int32, (VB, 128), 1).astype(F32)
    oh = (batchf[...] == lane).astype(F32)
    z_node = jnp.dot(oh, zp[...], preferred_element_type=F32)
    tn = jnp.dot(oh, tb[...], preferred_element_type=F32)
    temb = jnp.dot(_silu(tn * tw1[0:1, :] + tb1[0:1, :]), tw2[...],
                   preferred_element_type=F32) + tb2[0:1, :]
    h = (jnp.dot(z_node, cwz[...], preferred_element_type=F32)
         + jnp.dot(temb, cwt[...], preferred_element_type=F32) + cb[0:1, :])
    xv = xp[...]
    h_out[...] = h
    gd_out[:, :128] = jnp.dot(h, we1d[...], preferred_element_type=F32)
    gd_out[:, 128:] = xv
    gs_out[:, :128] = jnp.dot(h, we1s[...], preferred_element_type=F32)
    gs_out[:, 128:] = -xv


def _edge_body(last, r_ref, we2, be2, wx1, bx1, wx2r, bx2b, wd, be1, out_ref):
    rows = r_ref[...]
    diff = rows[:, 128:]
    dist2 = jnp.sum(diff * diff, axis=1, keepdims=True)
    pre = rows[:, :128] + dist2 * wd[0:1, :] + be1[0:1, :]
    u = _silu(pre)
    m = _silu(jnp.dot(u, we2[...], preferred_element_type=F32) + be2[0:1, :])
    c1 = _silu(jnp.dot(m, wx1[...], preferred_element_type=F32) + bx1[0:1, :])
    cw = jnp.sum(c1 * wx2r[0:1, :], axis=1, keepdims=True) + bx2b[0:1, 0:1]
    ones3 = (lax.broadcasted_iota(jnp.int32, (EB, DX), 1) == 3).astype(F32)
    xout = diff * cw + ones3
    if last:
        out_ref[...] = xout
    else:
        out_ref[:, :128] = m
        out_ref[:, 128:] = xout


def _node_body(p0, p1, h, xp, wh1h, wh1m, bh1, wh2, bh2, we1d, we1s,
               hn_out, xn_out, gd_out, gs_out):
    pa = p0[...]
    pb = p1[...]
    magg = pa[:, :128] + pb[:, :128]
    xs = pa[:, 128:] + pb[:, 128:]
    lane = lax.broadcasted_iota(jnp.int32, (VB, DX), 1)
    deg = jnp.sum(xs * (lane == 3).astype(F32), axis=1, keepdims=True)
    coef = 1.0 / jnp.maximum(deg, 1.0)
    xn = xp[...] + xs * coef * (lane < 3).astype(F32)
    hh = h[...]
    g = jnp.dot(hh, wh1h[...], preferred_element_type=F32) \
        + jnp.dot(magg, wh1m[...], preferred_element_type=F32) + bh1[0:1, :]
    hn = hh + jnp.dot(_silu(g), wh2[...], preferred_element_type=F32) + bh2[0:1, :]
    hn_out[...] = hn
    xn_out[...] = xn
    gd_out[:, :128] = jnp.dot(hn, we1d[...], preferred_element_type=F32)
    gd_out[:, 128:] = xn
    gs_out[:, :128] = jnp.dot(hn, we1s[...], preferred_element_type=F32)
    gs_out[:, 128:] = -xn


def _fin_body(p0, p1, xp, x0, vel_out):
    xs = p0[...] + p1[...]
    lane = lax.broadcasted_iota(jnp.int32, (VB, DX), 1)
    deg = jnp.sum(xs * (lane == 3).astype(F32), axis=1, keepdims=True)
    coef = 1.0 / jnp.maximum(deg, 1.0)
    vel_out[...] = xp[...] + xs * coef * (lane < 3).astype(F32) - x0[...]


def _wspec(shape):
    nd = len(shape)
    return pl.BlockSpec(shape, lambda i: (0,) * nd)


_init_call = pl.pallas_call(
    _init_body,
    grid=(VP // VB,),
    in_specs=[
        pl.BlockSpec((VB, 128), lambda i: (i, 0)),
        pl.BlockSpec((VB, DX), lambda i: (i, 0)),
        _wspec((128, LD)), _wspec((128, TD)),
        _wspec((8, TD)), _wspec((8, TD)), _wspec((TD, TD)), _wspec((8, TD)),
        _wspec((LD, HD)), _wspec((TD, HD)), _wspec((8, HD)),
        _wspec((HD, HD)), _wspec((HD, HD)),
    ],
    out_specs=[
        pl.BlockSpec((VB, HD), lambda i: (i, 0)),
        pl.BlockSpec((VB, DW), lambda i: (i, 0)),
        pl.BlockSpec((VB, DW), lambda i: (i, 0)),
    ],
    out_shape=[
        jax.ShapeDtypeStruct((VP, HD), F32),
        jax.ShapeDtypeStruct((VP, DW), F32),
        jax.ShapeDtypeStruct((VP, DW), F32),
    ],
)


def _make_edge(last):
    dwo = DX if last else DW
    return pl.pallas_call(
        functools.partial(_edge_body, last),
        grid=(EP // EB,),
        in_specs=[
            pl.BlockSpec((EB, DW), lambda i: (i, 0)),
            _wspec((HD, HD)), _wspec((8, HD)),
            _wspec((HD, HD)), _wspec((8, HD)),
            _wspec((8, HD)), _wspec((8, 8)),
            _wspec((8, HD)), _wspec((8, HD)),
        ],
        out_specs=pl.BlockSpec((EB, dwo), lambda i: (i, 0)),
        out_shape=jax.ShapeDtypeStruct((EP, dwo), F32),
    )


_edge_mid = _make_edge(False)
_edge_last = _make_edge(True)

_node_call = pl.pallas_call(
    _node_body,
    grid=(VP // VB,),
    in_specs=[
        pl.BlockSpec((VB, DW), lambda i: (i, 0)),
        pl.BlockSpec((VB, DW), lambda i: (i, 0)),
        pl.BlockSpec((VB, HD), lambda i: (i, 0)),
        pl.BlockSpec((VB, DX), lambda i: (i, 0)),
        _wspec((HD, HD)), _wspec((HD, HD)), _wspec((8, HD)),
        _wspec((HD, HD)), _wspec((8, HD)),
        _wspec((HD, HD)), _wspec((HD, HD)),
    ],
    out_specs=[
        pl.BlockSpec((VB, HD), lambda i: (i, 0)),
        pl.BlockSpec((VB, DX), lambda i: (i, 0)),
        pl.BlockSpec((VB, DW), lambda i: (i, 0)),
        pl.BlockSpec((VB, DW), lambda i: (i, 0)),
    ],
    out_shape=[
        jax.ShapeDtypeStruct((VP, HD), F32),
        jax.ShapeDtypeStruct((VP, DX), F32),
        jax.ShapeDtypeStruct((VP, DW), F32),
        jax.ShapeDtypeStruct((VP, DW), F32),
    ],
)

_fin_call = pl.pallas_call(
    _fin_body,
    grid=(VP // VB,),
    in_specs=[
        pl.BlockSpec((VB, DX), lambda i: (i, 0)),
        pl.BlockSpec((VB, DX), lambda i: (i, 0)),
        pl.BlockSpec((VB, DX), lambda i: (i, 0)),
        pl.BlockSpec((VB, DX), lambda i: (i, 0)),
    ],
    out_specs=pl.BlockSpec((VB, DX), lambda i: (i, 0)),
    out_shape=jax.ShapeDtypeStruct((VP, DX), F32),
)


def _bc8(v):
    v = v.reshape(-1)
    return jnp.broadcast_to(v[None, :], (8, v.shape[0]))


def kernel(pos, edge_index, batch, t, z, t_W1, t_b1, t_W2, t_b2, c_W, c_b,
           We1, be1, We2, be2, Wx1, bx1, Wx2, bx2, Wh1, bh1, Wh2, bh2,
           Wout, bout):
    src = edge_index[0].astype(jnp.int32)
    dst = edge_index[1].astype(jnp.int32)
    epad = jnp.full((EP - E,), V, jnp.int32)
    srcp = jnp.concatenate([src, epad])
    dstp = jnp.concatenate([dst, epad])

    xp0 = jnp.zeros((VP, DX), F32).at[:V, :3].set(pos)
    batchf = jnp.zeros((VP, 128), F32).at[:V, :].set(batch.astype(F32)[:, None])
    zp = jnp.zeros((128, LD), F32).at[:B].set(z)
    tb = jnp.zeros((128, TD), F32).at[:B].set(t[:, None])

    h, gd, gs = _init_call(
        batchf, xp0, zp, tb,
        _bc8(t_W1), _bc8(t_b1), t_W2, _bc8(t_b2),
        c_W[:LD], c_W[LD:], _bc8(c_b),
        We1[0, :HD], We1[0, HD:2 * HD],
    )

    xp = xp0
    velf = None
    for l in range(NL):
        r = _sc_gather(gd, gs, dstp, srcp)
        last = l == NL - 1
        edge_fn = _edge_last if last else _edge_mid
        w = edge_fn(
            r, We2[l], _bc8(be2[l]), Wx1[l], _bc8(bx1[l]),
            _bc8(Wx2[l]), jnp.broadcast_to(bx2[l].reshape(1, 1), (8, 8)),
            _bc8(We1[l, 2 * HD]), _bc8(be1[l]),
        )
        p = (_sc_scatter_x if last else _sc_scatter_full)(w, dstp)
        if last:
            velf = _fin_call(p[0], p[1], xp, xp0)
        else:
            h, xp, gd, gs = _node_call(
                p[0], p[1], h, xp,
                Wh1[l, :HD], Wh1[l, HD:], _bc8(bh1[l]), Wh2[l], _bc8(bh2[l]),
                We1[l + 1, :HD], We1[l + 1, HD:2 * HD],
            )
    return velf[:V, :3]


# speedup vs baseline: 2.4546x; 1.0914x over previous
import functools

import jax
import jax.numpy as jnp
from jax import lax
from jax.experimental import pallas as pl
from jax.experimental.pallas import tpu as pltpu
from jax.experimental.pallas import tpu_sc as plsc

V = 10000
E = 320000
B = 8
HD = 128
LD = 64
TD = 16
NL = 4

VP = 10240
EP = 327680
DW = 144
DX = 16
NW = 32
EPW = EP // NW
CH = 128
NCH = EPW // CH
RPT = VP // 16

EB = 2048
VB = 1024

F32 = jnp.float32


def _silu(x):
    return x * jax.nn.sigmoid(x)


_SC_MESH = plsc.VectorSubcoreMesh(
    core_axis_name="c", subcore_axis_name="s", num_cores=2, num_subcores=16)


def _sc_gather_body(eoff, epw, nch, gd, gs, dstp, srcp, out,
                    didx0, didx1, sidx0, sidx1,
                    bufa0, bufa1, bufb0, bufb1,
                    isem0, isem1, gsem0, gsem1, osem0, osem1):
    c = lax.axis_index("c")
    s = lax.axis_index("s")
    wid = s * 2 + c
    base_w = wid * epw
    didx = (didx0, didx1)
    sidx = (sidx0, sidx1)
    bufa = (bufa0, bufa1)
    bufb = (bufb0, bufb1)
    isem = (isem0, isem1)
    gsem = (gsem0, gsem1)
    osem = (osem0, osem1)

    def start_idx(i, b):
        base = eoff + base_w + i * CH
        pltpu.async_copy(dstp.at[pl.ds(base, CH)], didx[b], isem[b])
        pltpu.async_copy(srcp.at[pl.ds(base, CH)], sidx[b], isem[b])

    def wait_idx(i, b):
        base = eoff + base_w + i * CH
        pltpu.make_async_copy(dstp.at[pl.ds(base, CH)], didx[b], isem[b]).wait()
        pltpu.make_async_copy(srcp.at[pl.ds(base, CH)], sidx[b], isem[b]).wait()

    def start_gather(b):
        pltpu.async_copy(gd.at[didx[b]], bufa[b], gsem[b])
        pltpu.async_copy(gs.at[sidx[b]], bufb[b], gsem[b])

    def wait_gather(b):
        pltpu.make_async_copy(gd.at[didx[b]], bufa[b], gsem[b]).wait()
        pltpu.make_async_copy(gs.at[sidx[b]], bufb[b], gsem[b]).wait()

    def wait_out(i, b):
        pltpu.make_async_copy(
            bufa[b], out.at[pl.ds(base_w + i * CH, CH)], osem[b]).wait()

    start_idx(0, 0)
    wait_idx(0, 0)
    start_gather(0)
    start_idx(1, 1)

    def pair(p, carry):
        for b in range(2):
            i = 2 * p + b
            nb = 1 - b

            def _launch_next():
                wait_idx(i + 1, nb)
                if b == 0:
                    @pl.when(p >= 1)
                    def _wo():
                        wait_out(i - 1, nb)
                else:
                    wait_out(i - 1, nb)
                start_gather(nb)

            if b == 0:
                _launch_next()
            else:
                @pl.when(p < nch // 2 - 1)
                def _ln():
                    _launch_next()

            wait_gather(b)

            @pl.when(p < nch // 2 - 1)
            def _nidx():
                start_idx(i + 2, b)

            @plsc.parallel_loop(0, CH, 1, unroll=8)
            def _add(r):
                for j in range(DW // 16):
                    sl = pl.ds(j * 16, 16)
                    bufa[b][r, sl] = bufa[b][r, sl] + bufb[b][r, sl]

            pltpu.async_copy(bufa[b], out.at[pl.ds(base_w + i * CH, CH)],
                             osem[b])
        return carry

    lax.fori_loop(0, nch // 2, pair, 0)
    for b in range(2):
        wait_out(nch - 2 + b, b)


_SC_PARAMS = pltpu.CompilerParams(use_tc_tiling_on_sc=False)

NE = EP // 2
EPW2 = NE // NW
NCH2 = EPW2 // CH


def _make_gather(eoff):
    return pl.kernel(
        functools.partial(_sc_gather_body, eoff, EPW2, NCH2),
        out_type=jax.ShapeDtypeStruct((NE, DW), F32),
        mesh=_SC_MESH,
        compiler_params=_SC_PARAMS,
        scratch_types=[
            pltpu.VMEM((CH,), jnp.int32),
            pltpu.VMEM((CH,), jnp.int32),
            pltpu.VMEM((CH,), jnp.int32),
            pltpu.VMEM((CH,), jnp.int32),
            pltpu.VMEM((CH, DW), F32),
            pltpu.VMEM((CH, DW), F32),
            pltpu.VMEM((CH, DW), F32),
            pltpu.VMEM((CH, DW), F32),
            pltpu.SemaphoreType.DMA,
            pltpu.SemaphoreType.DMA,
            pltpu.SemaphoreType.DMA,
            pltpu.SemaphoreType.DMA,
            pltpu.SemaphoreType.DMA,
            pltpu.SemaphoreType.DMA,
        ],
    )


_sc_gather_a = _make_gather(0)
_sc_gather_b = _make_gather(NE)


def _sc_scatter_body(dw, eoff, w_hbm, dstp, out, idx0, idx1, buf0, buf1, acc,
                     lsem0, lsem1, ssem0, ssem1):
    c = lax.axis_index("c")
    s = lax.axis_index("s")
    wid = s * 2 + c
    base_w = wid * EPW2
    idxb = (idx0, idx1)
    buf = (buf0, buf1)
    lsem = (lsem0, lsem1)
    ssem = (ssem0, ssem1)

    def zrow(r, cc):
        for j in range(dw // 16):
            buf0[r, pl.ds(j * 16, 16)] = jnp.zeros((16,), F32)
        return cc

    lax.fori_loop(0, CH, zrow, 0)
    for k in range(RPT // CH):
        pltpu.sync_copy(buf0, acc.at[pl.ds(s * RPT + k * CH, CH)])
    plsc.subcore_barrier()

    def start_load(i, b):
        base = base_w + i * CH
        pltpu.async_copy(dstp.at[pl.ds(eoff + base, CH)], idxb[b], lsem[b])
        pltpu.async_copy(w_hbm.at[pl.ds(base, CH)], buf[b], lsem[b])

    def wait_load(i, b):
        base = base_w + i * CH
        pltpu.make_async_copy(
            dstp.at[pl.ds(eoff + base, CH)], idxb[b], lsem[b]).wait()
        pltpu.make_async_copy(w_hbm.at[pl.ds(base, CH)], buf[b], lsem[b]).wait()

    def wait_scat(b):
        pltpu.make_async_copy(buf[b], acc.at[idxb[b]], ssem[b]).wait()

    start_load(0, 0)

    def pair(p, carry):
        for b in range(2):
            i = 2 * p + b
            nb = 1 - b
            wait_load(i, b)

            def _next_load():
                wait_scat(nb)
                start_load(i + 1, nb)

            if b == 0:
                @pl.when(p >= 1)
                def _nl0():
                    _next_load()

                @pl.when(p == 0)
                def _nl1():
                    start_load(i + 1, nb)
            else:
                @pl.when(p < NCH2 // 2 - 1)
                def _nl2():
                    _next_load()

            pltpu.async_copy(buf[b], acc.at[idxb[b]], ssem[b], add=True)
        return carry

    lax.fori_loop(0, NCH2 // 2, pair, 0)
    wait_scat(0)
    wait_scat(1)
    plsc.subcore_barrier()
    for k in range(RPT // CH):
        r0 = s * RPT + k * CH
        pltpu.sync_copy(acc.at[pl.ds(r0, CH)], buf0)
        pltpu.sync_copy(buf0, out.at[c, pl.ds(r0, CH)])


def _make_scatter(dw, eoff):
    return pl.kernel(
        functools.partial(_sc_scatter_body, dw, eoff),
        out_type=jax.ShapeDtypeStruct((2, VP, dw), F32),
        mesh=_SC_MESH,
        compiler_params=_SC_PARAMS,
        scratch_types=[
            pltpu.VMEM((CH,), jnp.int32),
            pltpu.VMEM((CH,), jnp.int32),
            pltpu.VMEM((CH, dw), F32),
            pltpu.VMEM((CH, dw), F32),
            pltpu.VMEM_SHARED((VP, dw), F32),
            pltpu.SemaphoreType.DMA,
            pltpu.SemaphoreType.DMA,
            pltpu.SemaphoreType.DMA,
            pltpu.SemaphoreType.DMA,
        ],
    )


_sc_scatter_full_a = _make_scatter(DW, 0)
_sc_scatter_full_b = _make_scatter(DW, NE)
_sc_scatter_x_a = _make_scatter(DX, 0)
_sc_scatter_x_b = _make_scatter(DX, NE)


def _init_body(batchf, xp, zp, tb, tw1, tb1, tw2, tb2, cwz, cwt, cb,
               we1d, we1s, h_out, gd_out, gs_out):
    lane = lax.broadcasted_iota(jnp.int32, (VB, 128), 1).astype(F32)
    oh = (batchf[...] == lane).astype(F32)
    z_node = jnp.dot(oh, zp[...], preferred_element_type=F32)
    tn = jnp.dot(oh, tb[...], preferred_element_type=F32)
    temb = jnp.dot(_silu(tn * tw1[0:1, :] + tb1[0:1, :]), tw2[...],
                   preferred_element_type=F32) + tb2[0:1, :]
    h = (jnp.dot(z_node, cwz[...], preferred_element_type=F32)
         + jnp.dot(temb, cwt[...], preferred_element_type=F32) + cb[0:1, :])
    xv = xp[...]
    h_out[...] = h
    gd_out[:, :128] = jnp.dot(h, we1d[...], preferred_element_type=F32)
    gd_out[:, 128:] = xv
    gs_out[:, :128] = jnp.dot(h, we1s[...], preferred_element_type=F32)
    gs_out[:, 128:] = -xv


def _edge_body(last, r_ref, we2, be2, wx1, bx1, wx2r, bx2b, wd, be1, out_ref):
    rows = r_ref[...]
    diff = rows[:, 128:]
    dist2 = jnp.sum(diff * diff, axis=1, keepdims=True)
    pre = rows[:, :128] + dist2 * wd[0:1, :] + be1[0:1, :]
    u = _silu(pre)
    m = _silu(jnp.dot(u, we2[...], preferred_element_type=F32) + be2[0:1, :])
    c1 = _silu(jnp.dot(m, wx1[...], preferred_element_type=F32) + bx1[0:1, :])
    cw = jnp.sum(c1 * wx2r[0:1, :], axis=1, keepdims=True) + bx2b[0:1, 0:1]
    ones3 = (lax.broadcasted_iota(jnp.int32, (EB, DX), 1) == 3).astype(F32)
    xout = diff * cw + ones3
    if last:
        out_ref[...] = xout
    else:
        out_ref[:, :128] = m
        out_ref[:, 128:] = xout


def _node_body(p0, p1, p2, p3, h, xp, wh1h, wh1m, bh1, wh2, bh2, we1d, we1s,
               hn_out, xn_out, gd_out, gs_out):
    pa = (p0[...] + p1[...]) + (p2[...] + p3[...])
    magg = pa[:, :128]
    xs = pa[:, 128:]
    lane = lax.broadcasted_iota(jnp.int32, (VB, DX), 1)
    deg = jnp.sum(xs * (lane == 3).astype(F32), axis=1, keepdims=True)
    coef = 1.0 / jnp.maximum(deg, 1.0)
    xn = xp[...] + xs * coef * (lane < 3).astype(F32)
    hh = h[...]
    g = jnp.dot(hh, wh1h[...], preferred_element_type=F32) \
        + jnp.dot(magg, wh1m[...], preferred_element_type=F32) + bh1[0:1, :]
    hn = hh + jnp.dot(_silu(g), wh2[...], preferred_element_type=F32) + bh2[0:1, :]
    hn_out[...] = hn
    xn_out[...] = xn
    gd_out[:, :128] = jnp.dot(hn, we1d[...], preferred_element_type=F32)
    gd_out[:, 128:] = xn
    gs_out[:, :128] = jnp.dot(hn, we1s[...], preferred_element_type=F32)
    gs_out[:, 128:] = -xn


def _fin_body(p0, p1, p2, p3, xp, x0, vel_out):
    xs = (p0[...] + p1[...]) + (p2[...] + p3[...])
    lane = lax.broadcasted_iota(jnp.int32, (VB, DX), 1)
    deg = jnp.sum(xs * (lane == 3).astype(F32), axis=1, keepdims=True)
    coef = 1.0 / jnp.maximum(deg, 1.0)
    vel_out[...] = xp[...] + xs * coef * (lane < 3).astype(F32) - x0[...]


def _wspec(shape):
    nd = len(shape)
    return pl.BlockSpec(shape, lambda i: (0,) * nd)


_init_call = pl.pallas_call(
    _init_body,
    grid=(VP // VB,),
    in_specs=[
        pl.BlockSpec((VB, 128), lambda i: (i, 0)),
        pl.BlockSpec((VB, DX), lambda i: (i, 0)),
        _wspec((128, LD)), _wspec((128, TD)),
        _wspec((8, TD)), _wspec((8, TD)), _wspec((TD, TD)), _wspec((8, TD)),
        _wspec((LD, HD)), _wspec((TD, HD)), _wspec((8, HD)),
        _wspec((HD, HD)), _wspec((HD, HD)),
    ],
    out_specs=[
        pl.BlockSpec((VB, HD), lambda i: (i, 0)),
        pl.BlockSpec((VB, DW), lambda i: (i, 0)),
        pl.BlockSpec((VB, DW), lambda i: (i, 0)),
    ],
    out_shape=[
        jax.ShapeDtypeStruct((VP, HD), F32),
        jax.ShapeDtypeStruct((VP, DW), F32),
        jax.ShapeDtypeStruct((VP, DW), F32),
    ],
)


def _make_edge(last):
    dwo = DX if last else DW
    return pl.pallas_call(
        functools.partial(_edge_body, last),
        grid=(NE // EB,),
        in_specs=[
            pl.BlockSpec((EB, DW), lambda i: (i, 0)),
            _wspec((HD, HD)), _wspec((8, HD)),
            _wspec((HD, HD)), _wspec((8, HD)),
            _wspec((8, HD)), _wspec((8, 8)),
            _wspec((8, HD)), _wspec((8, HD)),
        ],
        out_specs=pl.BlockSpec((EB, dwo), lambda i: (i, 0)),
        out_shape=jax.ShapeDtypeStruct((NE, dwo), F32),
    )


_edge_mid = _make_edge(False)
_edge_last = _make_edge(True)

_node_call = pl.pallas_call(
    _node_body,
    grid=(VP // VB,),
    in_specs=[
        pl.BlockSpec((VB, DW), lambda i: (i, 0)),
        pl.BlockSpec((VB, DW), lambda i: (i, 0)),
        pl.BlockSpec((VB, DW), lambda i: (i, 0)),
        pl.BlockSpec((VB, DW), lambda i: (i, 0)),
        pl.BlockSpec((VB, HD), lambda i: (i, 0)),
        pl.BlockSpec((VB, DX), lambda i: (i, 0)),
        _wspec((HD, HD)), _wspec((HD, HD)), _wspec((8, HD)),
        _wspec((HD, HD)), _wspec((8, HD)),
        _wspec((HD, HD)), _wspec((HD, HD)),
    ],
    out_specs=[
        pl.BlockSpec((VB, HD), lambda i: (i, 0)),
        pl.BlockSpec((VB, DX), lambda i: (i, 0)),
        pl.BlockSpec((VB, DW), lambda i: (i, 0)),
        pl.BlockSpec((VB, DW), lambda i: (i, 0)),
    ],
    out_shape=[
        jax.ShapeDtypeStruct((VP, HD), F32),
        jax.ShapeDtypeStruct((VP, DX), F32),
        jax.ShapeDtypeStruct((VP, DW), F32),
        jax.ShapeDtypeStruct((VP, DW), F32),
    ],
)

_fin_call = pl.pallas_call(
    _fin_body,
    grid=(VP // VB,),
    in_specs=[
        pl.BlockSpec((VB, DX), lambda i: (i, 0)),
        pl.BlockSpec((VB, DX), lambda i: (i, 0)),
        pl.BlockSpec((VB, DX), lambda i: (i, 0)),
        pl.BlockSpec((VB, DX), lambda i: (i, 0)),
        pl.BlockSpec((VB, DX), lambda i: (i, 0)),
        pl.BlockSpec((VB, DX), lambda i: (i, 0)),
    ],
    out_specs=pl.BlockSpec((VB, DX), lambda i: (i, 0)),
    out_shape=jax.ShapeDtypeStruct((VP, DX), F32),
)


def _bc8(v):
    v = v.reshape(-1)
    return jnp.broadcast_to(v[None, :], (8, v.shape[0]))


def kernel(pos, edge_index, batch, t, z, t_W1, t_b1, t_W2, t_b2, c_W, c_b,
           We1, be1, We2, be2, Wx1, bx1, Wx2, bx2, Wh1, bh1, Wh2, bh2,
           Wout, bout):
    src = edge_index[0].astype(jnp.int32)
    dst = edge_index[1].astype(jnp.int32)
    epad = jnp.full((EP - E,), V, jnp.int32)
    srcp = jnp.concatenate([src, epad])
    dstp = jnp.concatenate([dst, epad])

    xp0 = jnp.zeros((VP, DX), F32).at[:V, :3].set(pos)
    batchf = jnp.zeros((VP, 128), F32).at[:V, :].set(batch.astype(F32)[:, None])
    zp = jnp.zeros((128, LD), F32).at[:B].set(z)
    tb = jnp.zeros((128, TD), F32).at[:B].set(t[:, None])

    h, gd, gs = _init_call(
        batchf, xp0, zp, tb,
        _bc8(t_W1), _bc8(t_b1), t_W2, _bc8(t_b2),
        c_W[:LD], c_W[LD:], _bc8(c_b),
        We1[0, :HD], We1[0, HD:2 * HD],
    )

    xp = xp0
    velf = None
    for l in range(NL):
        last = l == NL - 1
        edge_fn = _edge_last if last else _edge_mid
        ew = (We2[l], _bc8(be2[l]), Wx1[l], _bc8(bx1[l]),
              _bc8(Wx2[l]), jnp.broadcast_to(bx2[l].reshape(1, 1), (8, 8)),
              _bc8(We1[l, 2 * HD]), _bc8(be1[l]))
        ra = _sc_gather_a(gd, gs, dstp, srcp)
        wa = edge_fn(ra, *ew)
        rb = _sc_gather_b(gd, gs, dstp, srcp)
        wb = edge_fn(rb, *ew)
        if last:
            pa = _sc_scatter_x_a(wa, dstp)
            pb = _sc_scatter_x_b(wb, dstp)
            velf = _fin_call(pa[0], pa[1], pb[0], pb[1], xp, xp0)
        else:
            pa = _sc_scatter_full_a(wa, dstp)
            pb = _sc_scatter_full_b(wb, dstp)
            h, xp, gd, gs = _node_call(
                pa[0], pa[1], pb[0], pb[1], h, xp,
                Wh1[l, :HD], Wh1[l, HD:], _bc8(bh1[l]), Wh2[l], _bc8(bh2[l]),
                We1[l + 1, :HD], We1[l + 1, HD:2 * HD],
            )
    return velf[:V, :3]
